# baseline (speedup 1.0000x reference)
import jax
import jax.numpy as jnp
from jax.experimental import pallas as pl
from jax.experimental.pallas import tpu as pltpu

_VMEM_BUDGET = int(64 * 1024 * 1024 * 0.7)


def _largest_divisor_tile(total, unit_bytes, target_bytes):
    best = 1
    for t in range(1, total + 1):
        if total % t == 0 and t * unit_bytes <= target_bytes:
            best = t
    return best


def _mlp_gate(pooled, w1, b1, w2, b2):
    h = jax.lax.dot_general(
        pooled, w1, (((1,), (1,)), ((), ())),
        preferred_element_type=jnp.float32) + b1
    h = h * jax.nn.sigmoid(h)
    g = jax.lax.dot_general(
        h, w2, (((1,), (1,)), ((), ())),
        preferred_element_type=jnp.float32) + b2
    return jax.nn.sigmoid(g)


def _make_resident_kernel(n_tiles, tile, inv_hw):
    def se_kernel(x_ref, w1_ref, b1_ref, w2_ref, b2_ref, o_ref,
                  xs_ref, acc_ref, w1v, b1v, w2v, b2v, dma_sem):
        t = pl.program_id(1)

        @pl.when(t == 0)
        def _load_weights():
            for src, dst in ((w1_ref, w1v), (b1_ref, b1v),
                             (w2_ref, w2v), (b2_ref, b2v)):
                cp = pltpu.make_async_copy(src, dst, dma_sem)
                cp.start()
                cp.wait()

        @pl.when(t < n_tiles)
        def _pool_phase():
            x = x_ref[...]
            part = jnp.sum(x.astype(jnp.float32), axis=0)

            @pl.when(t == 0)
            def _init():
                acc_ref[...] = part

            @pl.when(t > 0)
            def _acc():
                acc_ref[...] += part

            xs_ref[pl.ds(t * tile, tile)] = x

        @pl.when(t == n_tiles - 1)
        def _excite():
            acc_ref[...] = _mlp_gate(acc_ref[...] * inv_hw, w1v[...],
                                     b1v[...][None, :], w2v[...],
                                     b2v[...][None, :])

        @pl.when(t >= n_tiles)
        def _scale_phase():
            j = t - n_tiles
            g = acc_ref[...].astype(o_ref.dtype)
            o_ref[...] = xs_ref[pl.ds(j * tile, tile)] * g[None]

    return se_kernel


def _resident_forward(xt, w1, b1r, w2, b2r, npar, t1):
    HW, N, C = xt.shape
    Cse = w1.shape[0]
    Np = N // npar
    n1 = HW // t1

    out = pl.pallas_call(
        _make_resident_kernel(n1, t1, 1.0 / float(HW)),
        out_shape=jax.ShapeDtypeStruct((HW, N, C), xt.dtype),
        grid=(npar, 2 * n1),
        in_specs=[
            pl.BlockSpec((t1, Np, C),
                         lambda p, t: (jnp.minimum(t, n1 - 1), p, 0)),
            pl.BlockSpec(memory_space=pl.ANY),
            pl.BlockSpec(memory_space=pl.ANY),
            pl.BlockSpec(memory_space=pl.ANY),
            pl.BlockSpec(memory_space=pl.ANY),
        ],
        out_specs=pl.BlockSpec((t1, Np, C),
                               lambda p, t: (jnp.maximum(t - n1, 0), p, 0)),
        scratch_shapes=[
            pltpu.VMEM((HW, Np, C), xt.dtype),
            pltpu.VMEM((Np, C), jnp.float32),
            pltpu.VMEM((Cse, C), jnp.float32),
            pltpu.VMEM((Cse,), jnp.float32),
            pltpu.VMEM((C, Cse), jnp.float32),
            pltpu.VMEM((C,), jnp.float32),
            pltpu.SemaphoreType.DMA,
        ],
        compiler_params=pltpu.CompilerParams(
            dimension_semantics=("parallel", "arbitrary"),
            vmem_limit_bytes=_VMEM_BUDGET),
    )(xt, w1, b1r, w2, b2r)
    return out


def _make_pool_kernel(num_tiles, inv_hw):
    def pool_kernel(x_ref, w1_ref, b1_ref, w2_ref, b2_ref, g_ref):
        t = pl.program_id(1)
        part = jnp.sum(x_ref[...].astype(jnp.float32), axis=0)

        @pl.when(t == 0)
        def _init():
            g_ref[...] = part

        @pl.when(t > 0)
        def _acc():
            g_ref[...] += part

        @pl.when(t == num_tiles - 1)
        def _excite():
            g_ref[...] = _mlp_gate(g_ref[...] * inv_hw, w1_ref[...],
                                   b1_ref[...], w2_ref[...], b2_ref[...])

    return pool_kernel


def _scale_kernel(x_ref, g_ref, o_ref):
    o_ref[...] = x_ref[...] * g_ref[...].astype(o_ref.dtype)[None]


def _two_pass_forward(xt, w1, b1r, w2, b2r, npar, t1):
    HW, N, C = xt.shape
    Cse = w1.shape[0]
    Np = N // npar
    n1 = HW // t1
    itemsize = jnp.dtype(xt.dtype).itemsize

    gate = pl.pallas_call(
        _make_pool_kernel(n1, 1.0 / float(HW)),
        out_shape=jax.ShapeDtypeStruct((N, C), jnp.float32),
        grid=(npar, n1),
        in_specs=[
            pl.BlockSpec((t1, Np, C), lambda p, t: (t, p, 0)),
            pl.BlockSpec((Cse, C), lambda p, t: (0, 0)),
            pl.BlockSpec((1, Cse), lambda p, t: (0, 0)),
            pl.BlockSpec((C, Cse), lambda p, t: (0, 0)),
            pl.BlockSpec((1, C), lambda p, t: (0, 0)),
        ],
        out_specs=pl.BlockSpec((Np, C), lambda p, t: (p, 0)),
        compiler_params=pltpu.CompilerParams(
            dimension_semantics=("parallel", "arbitrary"),
            vmem_limit_bytes=_VMEM_BUDGET),
    )(xt, w1, b1r, w2, b2r)

    t2 = _largest_divisor_tile(HW, N * C * itemsize, 4 << 20)
    nblk = HW // t2
    if nblk % npar == 0:
        n2 = nblk // npar
        sgrid = (npar, n2)
        x_spec = pl.BlockSpec((t2, N, C), lambda p, t: (p * n2 + t, 0, 0))
        g_spec = pl.BlockSpec((N, C), lambda p, t: (0, 0))
        o_spec = pl.BlockSpec((t2, N, C), lambda p, t: (p * n2 + t, 0, 0))
        sems = ("parallel", "parallel")
    else:
        sgrid = (nblk,)
        x_spec = pl.BlockSpec((t2, N, C), lambda t: (t, 0, 0))
        g_spec = pl.BlockSpec((N, C), lambda t: (0, 0))
        o_spec = pl.BlockSpec((t2, N, C), lambda t: (t, 0, 0))
        sems = ("parallel",)

    outt = pl.pallas_call(
        _scale_kernel,
        out_shape=jax.ShapeDtypeStruct((HW, N, C), xt.dtype),
        grid=sgrid,
        in_specs=[x_spec, g_spec],
        out_specs=o_spec,
        compiler_params=pltpu.CompilerParams(
            dimension_semantics=sems,
            vmem_limit_bytes=_VMEM_BUDGET),
    )(xt, gate)
    return outt


def kernel(x_nchw, w1, b1, w2, b2):
    N, C, H, W = x_nchw.shape
    Cse = w1.shape[0]
    HW = H * W
    itemsize = jnp.dtype(x_nchw.dtype).itemsize

    xt = jnp.transpose(x_nchw, (2, 3, 0, 1)).reshape(HW, N, C)

    npar = 2 if N % 16 == 0 else 1
    Np = N // npar
    slab = Np * C * itemsize
    t1 = _largest_divisor_tile(HW, slab, 3400 * 1024)

    resident_need = HW * slab + 4 * t1 * slab + (2 << 20)
    if resident_need <= _VMEM_BUDGET:
        outt = _resident_forward(xt, w1, b1, w2, b2, npar, t1)
    else:
        t1 = _largest_divisor_tile(HW, slab, 4 << 20)
        w1f = w1.astype(jnp.float32)
        w2f = w2.astype(jnp.float32)
        b1r = b1.reshape(1, Cse).astype(jnp.float32)
        b2r = b2.reshape(1, C).astype(jnp.float32)
        outt = _two_pass_forward(xt, w1f, b1r, w2f, b2r, npar, t1)

    return jnp.transpose(outt.reshape(H, W, N, C), (2, 3, 0, 1))


# speedup vs baseline: 1.0302x; 1.0302x over previous
import jax
import jax.numpy as jnp
from jax.experimental import pallas as pl
from jax.experimental.pallas import tpu as pltpu

_VMEM_BUDGET = int(64 * 1024 * 1024 * 0.7)


def _largest_divisor_tile(total, unit_bytes, target_bytes):
    best = 1
    for t in range(1, total + 1):
        if total % t == 0 and t * unit_bytes <= target_bytes:
            best = t
    return best


def _mlp_gate(pooled, w1, b1, w2, b2):
    h = jax.lax.dot_general(
        pooled, w1, (((1,), (1,)), ((), ())),
        preferred_element_type=jnp.float32) + b1
    h = h * jax.nn.sigmoid(h)
    g = jax.lax.dot_general(
        h, w2, (((1,), (1,)), ((), ())),
        preferred_element_type=jnp.float32) + b2
    return jax.nn.sigmoid(g)


def _make_resident_kernel(n_tiles, tile, inv_hw):
    def se_kernel(x_ref, w1_ref, b1_ref, w2_ref, b2_ref, o_ref,
                  xs_ref, acc_ref):
        t = pl.program_id(1)

        @pl.when(t < n_tiles)
        def _pool_phase():
            x = x_ref[...]
            part = jnp.sum(x.astype(jnp.float32), axis=0)

            @pl.when(t == 0)
            def _init():
                acc_ref[...] = part

            @pl.when(t > 0)
            def _acc():
                acc_ref[...] += part

            xs_ref[pl.ds(t * tile, tile)] = x

        @pl.when(t == n_tiles - 1)
        def _excite():
            acc_ref[...] = _mlp_gate(acc_ref[...] * inv_hw, w1_ref[...],
                                     b1_ref[...][None, :], w2_ref[...],
                                     b2_ref[...][None, :])

        @pl.when(t >= n_tiles)
        def _scale_phase():
            j = t - n_tiles
            g = acc_ref[...].astype(o_ref.dtype)
            o_ref[...] = xs_ref[pl.ds(j * tile, tile)] * g[None]

    return se_kernel


def _resident_forward(xt, w1, b1r, w2, b2r, npar, t1):
    HW, N, C = xt.shape
    Cse = w1.shape[0]
    Np = N // npar
    n1 = HW // t1

    out = pl.pallas_call(
        _make_resident_kernel(n1, t1, 1.0 / float(HW)),
        out_shape=jax.ShapeDtypeStruct((HW, N, C), xt.dtype),
        grid=(npar, 2 * n1),
        in_specs=[
            pl.BlockSpec((t1, Np, C),
                         lambda p, t: (jnp.minimum(t, n1 - 1), p, 0)),
            pl.BlockSpec((Cse, C), lambda p, t: (0, 0)),
            pl.BlockSpec((Cse,), lambda p, t: (0,)),
            pl.BlockSpec((C, Cse), lambda p, t: (0, 0)),
            pl.BlockSpec((C,), lambda p, t: (0,)),
        ],
        out_specs=pl.BlockSpec((t1, Np, C),
                               lambda p, t: (jnp.maximum(t - n1, 0), p, 0)),
        scratch_shapes=[
            pltpu.VMEM((HW, Np, C), xt.dtype),
            pltpu.VMEM((Np, C), jnp.float32),
        ],
        compiler_params=pltpu.CompilerParams(
            dimension_semantics=("parallel", "arbitrary"),
            vmem_limit_bytes=_VMEM_BUDGET),
    )(xt, w1, b1r, w2, b2r)
    return out


def _make_pool_kernel(num_tiles, inv_hw):
    def pool_kernel(x_ref, w1_ref, b1_ref, w2_ref, b2_ref, g_ref):
        t = pl.program_id(1)
        part = jnp.sum(x_ref[...].astype(jnp.float32), axis=0)

        @pl.when(t == 0)
        def _init():
            g_ref[...] = part

        @pl.when(t > 0)
        def _acc():
            g_ref[...] += part

        @pl.when(t == num_tiles - 1)
        def _excite():
            g_ref[...] = _mlp_gate(g_ref[...] * inv_hw, w1_ref[...],
                                   b1_ref[...], w2_ref[...], b2_ref[...])

    return pool_kernel


def _scale_kernel(x_ref, g_ref, o_ref):
    o_ref[...] = x_ref[...] * g_ref[...].astype(o_ref.dtype)[None]


def _two_pass_forward(xt, w1, b1r, w2, b2r, npar, t1):
    HW, N, C = xt.shape
    Cse = w1.shape[0]
    Np = N // npar
    n1 = HW // t1
    itemsize = jnp.dtype(xt.dtype).itemsize

    gate = pl.pallas_call(
        _make_pool_kernel(n1, 1.0 / float(HW)),
        out_shape=jax.ShapeDtypeStruct((N, C), jnp.float32),
        grid=(npar, n1),
        in_specs=[
            pl.BlockSpec((t1, Np, C), lambda p, t: (t, p, 0)),
            pl.BlockSpec((Cse, C), lambda p, t: (0, 0)),
            pl.BlockSpec((1, Cse), lambda p, t: (0, 0)),
            pl.BlockSpec((C, Cse), lambda p, t: (0, 0)),
            pl.BlockSpec((1, C), lambda p, t: (0, 0)),
        ],
        out_specs=pl.BlockSpec((Np, C), lambda p, t: (p, 0)),
        compiler_params=pltpu.CompilerParams(
            dimension_semantics=("parallel", "arbitrary"),
            vmem_limit_bytes=_VMEM_BUDGET),
    )(xt, w1, b1r, w2, b2r)

    t2 = _largest_divisor_tile(HW, N * C * itemsize, 4 << 20)
    nblk = HW // t2
    if nblk % npar == 0:
        n2 = nblk // npar
        sgrid = (npar, n2)
        x_spec = pl.BlockSpec((t2, N, C), lambda p, t: (p * n2 + t, 0, 0))
        g_spec = pl.BlockSpec((N, C), lambda p, t: (0, 0))
        o_spec = pl.BlockSpec((t2, N, C), lambda p, t: (p * n2 + t, 0, 0))
        sems = ("parallel", "parallel")
    else:
        sgrid = (nblk,)
        x_spec = pl.BlockSpec((t2, N, C), lambda t: (t, 0, 0))
        g_spec = pl.BlockSpec((N, C), lambda t: (0, 0))
        o_spec = pl.BlockSpec((t2, N, C), lambda t: (t, 0, 0))
        sems = ("parallel",)

    outt = pl.pallas_call(
        _scale_kernel,
        out_shape=jax.ShapeDtypeStruct((HW, N, C), xt.dtype),
        grid=sgrid,
        in_specs=[x_spec, g_spec],
        out_specs=o_spec,
        compiler_params=pltpu.CompilerParams(
            dimension_semantics=sems,
            vmem_limit_bytes=_VMEM_BUDGET),
    )(xt, gate)
    return outt


def kernel(x_nchw, w1, b1, w2, b2):
    N, C, H, W = x_nchw.shape
    Cse = w1.shape[0]
    HW = H * W
    itemsize = jnp.dtype(x_nchw.dtype).itemsize

    xt = jnp.transpose(x_nchw, (2, 3, 0, 1)).reshape(HW, N, C)

    npar = 2 if N % 16 == 0 else 1
    Np = N // npar
    slab = Np * C * itemsize
    t1 = _largest_divisor_tile(HW, slab, 4 << 20)

    resident_need = HW * slab + 4 * t1 * slab + (2 << 20)
    if resident_need <= _VMEM_BUDGET:
        outt = _resident_forward(xt, w1, b1, w2, b2, npar, t1)
    else:
        t1 = _largest_divisor_tile(HW, slab, 4 << 20)
        w1f = w1.astype(jnp.float32)
        w2f = w2.astype(jnp.float32)
        b1r = b1.reshape(1, Cse).astype(jnp.float32)
        b2r = b2.reshape(1, C).astype(jnp.float32)
        outt = _two_pass_forward(xt, w1f, b1r, w2f, b2r, npar, t1)

    return jnp.transpose(outt.reshape(H, W, N, C), (2, 3, 0, 1))
